# Initial kernel scaffold; baseline (speedup 1.0000x reference)
import jax
import jax.numpy as jnp
from jax.experimental import pallas as pl


def kernel(x_node_h, x_global_features, edge_index, batch_idx, W1, as1, ad1, b1, W2, as2, ad2, b2, W3, as3, ad3, b3, Wm1, bm1, Wm2, bm2, Wm3, bm3, Wm4, bm4):
    raise NotImplementedError("write your pallas kernel here")


# speedup vs baseline: 27.2648x; 27.2648x over previous
import dataclasses
import functools

import jax
import jax.numpy as jnp
from jax import lax
from jax.experimental import pallas as pl
from jax.experimental.pallas import tpu as pltpu
from jax.experimental.pallas import tpu_sc as plsc

N = 50000
E = 800000
B = 50
H = 4
C = 16
F = 64
G = 2
FG = 32
AW = 36
BN = 2000
NB = N // BN
BPAD = 56
NSUB = 16
EPT = E // NSUB
EK = 80
NCH = EPT // EK
NPT_A = 3128
NPT_B = N - 15 * NPT_A

_f32 = jnp.float32


def _stage0_body(xn_ref, bidx_ref, glob_ref, w_ref, a_ref,
                 ht_ref, as_ref, ad_ref):
    xn = xn_ref[...]
    bidx = bidx_ref[...][:, 0]
    onehot = (bidx[:, None] ==
              lax.broadcasted_iota(jnp.int32, (BN, B), 1)).astype(_f32)
    w = w_ref[...]
    gw = jnp.dot(glob_ref[...], w[32:64, :], preferred_element_type=_f32)
    h = (jnp.dot(xn, w[0:32, :], preferred_element_type=_f32)
         + jnp.dot(onehot, gw, preferred_element_type=_f32))
    a8 = jnp.dot(h, a_ref[...], preferred_element_type=_f32)
    zpad = jnp.zeros((BN, 12), _f32)
    ht_ref[0] = h[:, 0:FG]
    ht_ref[1] = h[:, FG:F]
    as_ref[...] = jnp.concatenate([zpad, a8[:, 0:4]], axis=1)
    ad_ref[...] = jnp.concatenate([zpad, a8[:, 4:8]], axis=1)


def _stage0(xn, bidx2, glob, w1, a1):
    return pl.pallas_call(
        _stage0_body,
        grid=(NB,),
        in_specs=[
            pl.BlockSpec((BN, 32), lambda i: (i, 0)),
            pl.BlockSpec((BN, 1), lambda i: (i, 0)),
            pl.BlockSpec((B, 32), lambda i: (0, 0)),
            pl.BlockSpec((F, F), lambda i: (0, 0)),
            pl.BlockSpec((F, 8), lambda i: (0, 0)),
        ],
        out_specs=[
            pl.BlockSpec((G, BN, FG), lambda i: (0, i, 0)),
            pl.BlockSpec((BN, 16), lambda i: (i, 0)),
            pl.BlockSpec((BN, 16), lambda i: (i, 0)),
        ],
        out_shape=[
            jax.ShapeDtypeStruct((G, N, FG), _f32),
            jax.ShapeDtypeStruct((N, 16), _f32),
            jax.ShapeDtypeStruct((N, 16), _f32),
        ],
    )(xn, bidx2, glob, w1, a1)


def _mid_body(acc_ref, r_ref, b_ref, w_ref, a_ref, ht_ref, as_ref, ad_ref):
    m0 = acc_ref[0]
    m1 = acc_ref[1]
    den = m0[:, FG:AW]
    dd = jnp.dot(den, r_ref[...], preferred_element_type=_f32)
    m = jnp.concatenate([m0[:, 0:FG], m1[:, 0:FG]], axis=1)
    x = m / (dd + 1e-16) + b_ref[...]
    x = jnp.maximum(x, 0.0)
    h = jnp.dot(x, w_ref[...], preferred_element_type=_f32)
    a8 = jnp.dot(h, a_ref[...], preferred_element_type=_f32)
    zpad = jnp.zeros((BN, 12), _f32)
    ht_ref[0] = h[:, 0:FG]
    ht_ref[1] = h[:, FG:F]
    as_ref[...] = jnp.concatenate([zpad, a8[:, 0:4]], axis=1)
    ad_ref[...] = jnp.concatenate([zpad, a8[:, 4:8]], axis=1)


def _mid(acc3, r, b, w, a):
    return pl.pallas_call(
        _mid_body,
        grid=(NB,),
        in_specs=[
            pl.BlockSpec((G, BN, AW), lambda i: (0, i, 0)),
            pl.BlockSpec((4, F), lambda i: (0, 0)),
            pl.BlockSpec((1, F), lambda i: (0, 0)),
            pl.BlockSpec((F, F), lambda i: (0, 0)),
            pl.BlockSpec((F, 8), lambda i: (0, 0)),
        ],
        out_specs=[
            pl.BlockSpec((G, BN, FG), lambda i: (0, i, 0)),
            pl.BlockSpec((BN, 16), lambda i: (i, 0)),
            pl.BlockSpec((BN, 16), lambda i: (i, 0)),
        ],
        out_shape=[
            jax.ShapeDtypeStruct((G, N, FG), _f32),
            jax.ShapeDtypeStruct((N, 16), _f32),
            jax.ShapeDtypeStruct((N, 16), _f32),
        ],
    )(acc3, r, b, w, a)


def _stage3_body(acc_ref, r_ref, b_ref, bidx_ref,
                 wm1_ref, bm1_ref, wm2_ref, bm2_ref,
                 wm3_ref, bm3_ref, wm4_ref, bm4_ref,
                 out_ref, sum_ref, cnt_ref):
    i = pl.program_id(0)

    @pl.when(i == 0)
    def _():
        sum_ref[...] = jnp.zeros((BPAD, F), _f32)
        cnt_ref[...] = jnp.zeros((BPAD, F), _f32)
        out_ref[...] = jnp.zeros((BPAD, 1), _f32)

    m0 = acc_ref[0]
    m1 = acc_ref[1]
    den = m0[:, FG:AW]
    dd = jnp.dot(den, r_ref[...], preferred_element_type=_f32)
    m = jnp.concatenate([m0[:, 0:FG], m1[:, 0:FG]], axis=1)
    x = m / (dd + 1e-16) + b_ref[...]

    bidx = bidx_ref[...][:, 0]
    onehot = (bidx[:, None] ==
              lax.broadcasted_iota(jnp.int32, (BN, BPAD), 1)).astype(_f32)
    dn = (((0,), (0,)), ((), ()))
    sum_ref[...] += lax.dot_general(onehot, x, dn,
                                    preferred_element_type=_f32)
    cnt_ref[...] += lax.dot_general(onehot, jnp.ones((BN, F), _f32), dn,
                                    preferred_element_type=_f32)

    @pl.when(i == NB - 1)
    def _():
        g = sum_ref[...] / jnp.maximum(cnt_ref[...], 1.0)
        h1 = jnp.maximum(jnp.dot(g, wm1_ref[...],
                                 preferred_element_type=_f32)
                         + bm1_ref[...], 0.0)
        h2 = jnp.maximum(jnp.dot(h1, wm2_ref[...],
                                 preferred_element_type=_f32)
                         + bm2_ref[...], 0.0)
        h3 = jnp.maximum(jnp.dot(h2, wm3_ref[...],
                                 preferred_element_type=_f32)
                         + bm3_ref[...], 0.0)
        out_ref[...] = (jnp.dot(h3, wm4_ref[...],
                                preferred_element_type=_f32)
                        + bm4_ref[...])


def _stage3(acc3, r, b, bidx2, wm1, bm1, wm2, bm2, wm3, bm3, wm4, bm4):
    full = lambda s: pl.BlockSpec(s, lambda i: tuple(0 for _ in s))
    return pl.pallas_call(
        _stage3_body,
        grid=(NB,),
        in_specs=[
            pl.BlockSpec((G, BN, AW), lambda i: (0, i, 0)),
            full((4, F)),
            full((1, F)),
            pl.BlockSpec((BN, 1), lambda i: (i, 0)),
            full((F, F)), full((1, F)),
            full((F, F)), full((1, F)),
            full((F, F)), full((1, F)),
            full((F, 1)), full((1, 1)),
        ],
        out_specs=pl.BlockSpec((BPAD, 1), lambda i: (0, 0)),
        out_shape=jax.ShapeDtypeStruct((BPAD, 1), _f32),
        scratch_shapes=[
            pltpu.VMEM((BPAD, F), _f32),
            pltpu.VMEM((BPAD, F), _f32),
        ],
    )(acc3, r, b, bidx2, wm1, bm1, wm2, bm2, wm3, bm3, wm4, bm4)


def _sc_body(ht_hbm, as_hbm, ad_hbm, src_hbm, dst_hbm, z_hbm,
             out_hbm,
             spm, srcv, dstv, hidx, asb, adb, hb, msgb):
    g = lax.axis_index("c")
    s = lax.axis_index("s")
    gN = g * N
    lane = lax.iota(jnp.int32, 16)
    m0 = (lane == 12 + 2 * g).astype(_f32)
    m1 = (lane == 13 + 2 * g).astype(_f32)

    @pl.when(s < 15)
    def _():
        pltpu.sync_copy(z_hbm, spm.at[pl.ds(s * NPT_A, NPT_A)])

    @pl.when(s == 15)
    def _():
        pltpu.sync_copy(z_hbm.at[pl.ds(0, NPT_B)],
                        spm.at[pl.ds(15 * NPT_A, NPT_B)])

    plsc.subcore_barrier()

    @pl.loop(0, NCH)
    def _chunk(ci):
        off = s * EPT + ci * EK
        pltpu.sync_copy(src_hbm.at[pl.ds(off, EK)], srcv)
        pltpu.sync_copy(dst_hbm.at[pl.ds(off, EK)], dstv)

        @pl.loop(0, EK, step=16)
        def _(i):
            hidx[pl.ds(i, 16)] = srcv[pl.ds(i, 16)] + gN

        pltpu.sync_copy(as_hbm.at[srcv], asb)
        pltpu.sync_copy(ad_hbm.at[dstv], adb)
        pltpu.sync_copy(ht_hbm.at[hidx], hb)

        @pl.loop(0, EK)
        def _(e):
            t = asb[e, 0:16] + adb[e, 0:16]
            ex = jnp.exp(jnp.maximum(t, 0.2 * t))
            msgb[e, 20:36] = ex
            e0 = jnp.sum(ex * m0)
            e1 = jnp.sum(ex * m1)
            msgb[e, 0:16] = hb[e, 0:16] * e0
            msgb[e, 16:32] = hb[e, 16:32] * e1

        pltpu.sync_copy(msgb, spm.at[dstv], add=True)

    plsc.subcore_barrier()

    @pl.when(s < 15)
    def _():
        pltpu.sync_copy(spm.at[pl.ds(s * NPT_A, NPT_A)],
                        out_hbm.at[pl.ds(gN + s * NPT_A, NPT_A)])

    @pl.when(s == 15)
    def _():
        pltpu.sync_copy(spm.at[pl.ds(15 * NPT_A, NPT_B)],
                        out_hbm.at[pl.ds(gN + 15 * NPT_A, NPT_B)])


@functools.partial(jax.jit, static_argnums=())
def _sc_edge_pass(ht2, as_t, ad_t, src, dst, z36):
    mesh = plsc.VectorSubcoreMesh(core_axis_name="c", subcore_axis_name="s")
    cp = pltpu.CompilerParams(needs_layout_passes=False,
                              use_tc_tiling_on_sc=False)
    k = pl.kernel(
        _sc_body,
        out_type=jax.ShapeDtypeStruct((G * N, AW), _f32),
        mesh=mesh,
        compiler_params=cp,
        scratch_types=[
            pltpu.VMEM_SHARED((N, AW), _f32),
            pltpu.VMEM((EK,), jnp.int32),
            pltpu.VMEM((EK,), jnp.int32),
            pltpu.VMEM((EK,), jnp.int32),
            pltpu.VMEM((EK, 16), _f32),
            pltpu.VMEM((EK, 16), _f32),
            pltpu.VMEM((EK, FG), _f32),
            pltpu.VMEM((EK, AW), _f32),
        ],
    )
    return k(ht2, as_t, ad_t, src, dst, z36)


def _att_mat(a_s, a_d):
    eye = jnp.eye(H, dtype=_f32)
    mk = lambda a: (eye[:, None, :] * a[:, :, None]).reshape(F, H)
    return jnp.concatenate([mk(a_s), mk(a_d)], axis=1)


def kernel(x_node_h, x_global_features, edge_index, batch_idx,
           W1, as1, ad1, b1, W2, as2, ad2, b2, W3, as3, ad3, b3,
           Wm1, bm1, Wm2, bm2, Wm3, bm3, Wm4, bm4):
    src = edge_index[0]
    dst = edge_index[1]
    bidx2 = batch_idx.reshape(N, 1)
    z36 = jnp.zeros((NPT_A, AW), _f32)
    r = jnp.repeat(jnp.eye(4, dtype=_f32), C, axis=1)

    a1 = _att_mat(as1, ad1)
    a2 = _att_mat(as2, ad2)
    a3 = _att_mat(as3, ad3)

    ht, as_t, ad_t = _stage0(x_node_h, bidx2, x_global_features, W1, a1)
    acc = _sc_edge_pass(ht.reshape(G * N, FG), as_t, ad_t, src, dst, z36)

    ht, as_t, ad_t = _mid(acc.reshape(G, N, AW), r, b1.reshape(1, F), W2, a2)
    acc = _sc_edge_pass(ht.reshape(G * N, FG), as_t, ad_t, src, dst, z36)

    ht, as_t, ad_t = _mid(acc.reshape(G, N, AW), r, b2.reshape(1, F), W3, a3)
    acc = _sc_edge_pass(ht.reshape(G * N, FG), as_t, ad_t, src, dst, z36)

    out = _stage3(acc.reshape(G, N, AW), r, b3.reshape(1, F), bidx2,
                  Wm1, bm1.reshape(1, F), Wm2, bm2.reshape(1, F),
                  Wm3, bm3.reshape(1, F), Wm4, bm4.reshape(1, 1))
    return out[:B, 0]


# speedup vs baseline: 54.2053x; 1.9881x over previous
import dataclasses
import functools

import jax
import jax.numpy as jnp
from jax import lax
from jax.experimental import pallas as pl
from jax.experimental.pallas import tpu as pltpu
from jax.experimental.pallas import tpu_sc as plsc

N = 50000
E = 800000
B = 50
H = 4
C = 16
F = 64
G = 2
FG = 32
AW = 36
BN = 2000
NB = N // BN
BPAD = 56
NSUB = 16
EPT = E // NSUB
EK = 80
NCH = EPT // EK
NPT_A = 3128
NPT_B = N - 15 * NPT_A

_f32 = jnp.float32


def _stage0_body(xn_ref, bidx_ref, glob_ref, w_ref, a_ref,
                 ht_ref, as_ref, ad_ref):
    xn = xn_ref[...]
    bidx = bidx_ref[...][:, 0]
    onehot = (bidx[:, None] ==
              lax.broadcasted_iota(jnp.int32, (BN, B), 1)).astype(_f32)
    w = w_ref[...]
    gw = jnp.dot(glob_ref[...], w[32:64, :], preferred_element_type=_f32)
    h = (jnp.dot(xn, w[0:32, :], preferred_element_type=_f32)
         + jnp.dot(onehot, gw, preferred_element_type=_f32))
    a8 = jnp.dot(h, a_ref[...], preferred_element_type=_f32)
    zpad = jnp.zeros((BN, 12), _f32)
    ht_ref[0] = h[:, 0:FG]
    ht_ref[1] = h[:, FG:F]
    as_ref[...] = jnp.concatenate([zpad, a8[:, 0:4]], axis=1)
    ad_ref[...] = jnp.concatenate([zpad, a8[:, 4:8]], axis=1)


def _stage0(xn, bidx2, glob, w1, a1):
    return pl.pallas_call(
        _stage0_body,
        grid=(NB,),
        in_specs=[
            pl.BlockSpec((BN, 32), lambda i: (i, 0)),
            pl.BlockSpec((BN, 1), lambda i: (i, 0)),
            pl.BlockSpec((B, 32), lambda i: (0, 0)),
            pl.BlockSpec((F, F), lambda i: (0, 0)),
            pl.BlockSpec((F, 8), lambda i: (0, 0)),
        ],
        out_specs=[
            pl.BlockSpec((G, BN, FG), lambda i: (0, i, 0)),
            pl.BlockSpec((BN, 16), lambda i: (i, 0)),
            pl.BlockSpec((BN, 16), lambda i: (i, 0)),
        ],
        out_shape=[
            jax.ShapeDtypeStruct((G, N, FG), _f32),
            jax.ShapeDtypeStruct((N, 16), _f32),
            jax.ShapeDtypeStruct((N, 16), _f32),
        ],
    )(xn, bidx2, glob, w1, a1)


def _mid_body(acc_ref, r_ref, b_ref, w_ref, a_ref, ht_ref, as_ref, ad_ref):
    m0 = acc_ref[0]
    m1 = acc_ref[1]
    den = m0[:, FG:AW]
    dd = jnp.dot(den, r_ref[...], preferred_element_type=_f32)
    m = jnp.concatenate([m0[:, 0:FG], m1[:, 0:FG]], axis=1)
    x = m / (dd + 1e-16) + b_ref[...]
    x = jnp.maximum(x, 0.0)
    h = jnp.dot(x, w_ref[...], preferred_element_type=_f32)
    a8 = jnp.dot(h, a_ref[...], preferred_element_type=_f32)
    zpad = jnp.zeros((BN, 12), _f32)
    ht_ref[0] = h[:, 0:FG]
    ht_ref[1] = h[:, FG:F]
    as_ref[...] = jnp.concatenate([zpad, a8[:, 0:4]], axis=1)
    ad_ref[...] = jnp.concatenate([zpad, a8[:, 4:8]], axis=1)


def _mid(acc3, r, b, w, a):
    return pl.pallas_call(
        _mid_body,
        grid=(NB,),
        in_specs=[
            pl.BlockSpec((G, BN, AW), lambda i: (0, i, 0)),
            pl.BlockSpec((4, F), lambda i: (0, 0)),
            pl.BlockSpec((1, F), lambda i: (0, 0)),
            pl.BlockSpec((F, F), lambda i: (0, 0)),
            pl.BlockSpec((F, 8), lambda i: (0, 0)),
        ],
        out_specs=[
            pl.BlockSpec((G, BN, FG), lambda i: (0, i, 0)),
            pl.BlockSpec((BN, 16), lambda i: (i, 0)),
            pl.BlockSpec((BN, 16), lambda i: (i, 0)),
        ],
        out_shape=[
            jax.ShapeDtypeStruct((G, N, FG), _f32),
            jax.ShapeDtypeStruct((N, 16), _f32),
            jax.ShapeDtypeStruct((N, 16), _f32),
        ],
    )(acc3, r, b, w, a)


def _stage3_body(acc_ref, r_ref, b_ref, bidx_ref,
                 wm1_ref, bm1_ref, wm2_ref, bm2_ref,
                 wm3_ref, bm3_ref, wm4_ref, bm4_ref,
                 out_ref, sum_ref, cnt_ref):
    i = pl.program_id(0)

    @pl.when(i == 0)
    def _():
        sum_ref[...] = jnp.zeros((BPAD, F), _f32)
        cnt_ref[...] = jnp.zeros((BPAD, F), _f32)
        out_ref[...] = jnp.zeros((BPAD, 1), _f32)

    m0 = acc_ref[0]
    m1 = acc_ref[1]
    den = m0[:, FG:AW]
    dd = jnp.dot(den, r_ref[...], preferred_element_type=_f32)
    m = jnp.concatenate([m0[:, 0:FG], m1[:, 0:FG]], axis=1)
    x = m / (dd + 1e-16) + b_ref[...]

    bidx = bidx_ref[...][:, 0]
    onehot = (bidx[:, None] ==
              lax.broadcasted_iota(jnp.int32, (BN, BPAD), 1)).astype(_f32)
    dn = (((0,), (0,)), ((), ()))
    sum_ref[...] += lax.dot_general(onehot, x, dn,
                                    preferred_element_type=_f32)
    cnt_ref[...] += lax.dot_general(onehot, jnp.ones((BN, F), _f32), dn,
                                    preferred_element_type=_f32)

    @pl.when(i == NB - 1)
    def _():
        g = sum_ref[...] / jnp.maximum(cnt_ref[...], 1.0)
        h1 = jnp.maximum(jnp.dot(g, wm1_ref[...],
                                 preferred_element_type=_f32)
                         + bm1_ref[...], 0.0)
        h2 = jnp.maximum(jnp.dot(h1, wm2_ref[...],
                                 preferred_element_type=_f32)
                         + bm2_ref[...], 0.0)
        h3 = jnp.maximum(jnp.dot(h2, wm3_ref[...],
                                 preferred_element_type=_f32)
                         + bm3_ref[...], 0.0)
        out_ref[...] = (jnp.dot(h3, wm4_ref[...],
                                preferred_element_type=_f32)
                        + bm4_ref[...])


def _stage3(acc3, r, b, bidx2, wm1, bm1, wm2, bm2, wm3, bm3, wm4, bm4):
    full = lambda s: pl.BlockSpec(s, lambda i: tuple(0 for _ in s))
    return pl.pallas_call(
        _stage3_body,
        grid=(NB,),
        in_specs=[
            pl.BlockSpec((G, BN, AW), lambda i: (0, i, 0)),
            full((4, F)),
            full((1, F)),
            pl.BlockSpec((BN, 1), lambda i: (i, 0)),
            full((F, F)), full((1, F)),
            full((F, F)), full((1, F)),
            full((F, F)), full((1, F)),
            full((F, 1)), full((1, 1)),
        ],
        out_specs=pl.BlockSpec((BPAD, 1), lambda i: (0, 0)),
        out_shape=jax.ShapeDtypeStruct((BPAD, 1), _f32),
        scratch_shapes=[
            pltpu.VMEM((BPAD, F), _f32),
            pltpu.VMEM((BPAD, F), _f32),
        ],
    )(acc3, r, b, bidx2, wm1, bm1, wm2, bm2, wm3, bm3, wm4, bm4)


def _sc_body(ht_hbm, as_hbm, ad_hbm, src_hbm, dst_hbm, z_hbm,
             out_hbm,
             spm,
             sv0, dv0, sv1, dv1, sv2, dv2, sv3, dv3,
             hx0, hx1,
             ab0, db0, hb0, mb0,
             ab1, db1, hb1, mb1,
             si0, si1, si2, si3, sg0, sg1):
    g = lax.axis_index("c")
    s = lax.axis_index("s")
    gN = g * N
    lane = lax.iota(jnp.int32, 16)
    m0v = (lane == 12 + 2 * g).astype(_f32)
    m1v = (lane == 13 + 2 * g).astype(_f32)

    @pl.when(s < 15)
    def _():
        pltpu.sync_copy(z_hbm, spm.at[pl.ds(s * NPT_A, NPT_A)])

    @pl.when(s == 15)
    def _():
        pltpu.sync_copy(z_hbm.at[pl.ds(0, NPT_B)],
                        spm.at[pl.ds(15 * NPT_A, NPT_B)])

    plsc.subcore_barrier()

    sv = (sv0, sv1, sv2, sv3)
    dv = (dv0, dv1, dv2, dv3)
    si = (si0, si1, si2, si3)
    hx = (hx0, hx1)
    ab = (ab0, ab1)
    db = (db0, db1)
    hb = (hb0, hb1)
    mb = (mb0, mb1)
    sg = (sg0, sg1)

    def idx_pair(i, r):
        off = s * EPT + i * EK
        return (pltpu.make_async_copy(src_hbm.at[pl.ds(off, EK)], sv[r], si[r]),
                pltpu.make_async_copy(dst_hbm.at[pl.ds(off, EK)], dv[r], si[r]))

    def L(i, r):
        c1, c2 = idx_pair(i, r)
        c1.start()
        c2.start()

    def gath(r, b):
        return (pltpu.make_async_copy(as_hbm.at[sv[r]], ab[b], sg[b]),
                pltpu.make_async_copy(ad_hbm.at[dv[r]], db[b], sg[b]),
                pltpu.make_async_copy(ht_hbm.at[hx[b]], hb[b], sg[b]))

    def G(i, r, b):
        c1, c2 = idx_pair(i, r)
        c1.wait()
        c2.wait()

        @pl.loop(0, EK, step=16)
        def _(j):
            hx[b][pl.ds(j, 16)] = sv[r][pl.ds(j, 16)] + gN

        g1, g2, g3 = gath(r, b)
        g1.start()
        g2.start()
        g3.start()

    def C(r, b):
        g1, g2, g3 = gath(r, b)
        g1.wait()
        g2.wait()
        g3.wait()

        @pl.loop(0, EK)
        def _(e):
            t = ab[b][e, 0:16] + db[b][e, 0:16]
            ex = jnp.exp(jnp.maximum(t, 0.2 * t))
            mb[b][e, 20:36] = ex
            e0 = jnp.sum(ex * m0v)
            e1 = jnp.sum(ex * m1v)
            mb[b][e, 0:16] = hb[b][e, 0:16] * e0
            mb[b][e, 16:32] = hb[b][e, 16:32] * e1

        pltpu.sync_copy(mb[b], spm.at[dv[r]], add=True)

    L(0, 0)
    L(1, 1)
    L(2, 2)
    L(3, 3)
    G(0, 0, 0)

    @pl.loop(0, NCH // 4 - 1)
    def _(k):
        c = 4 * k
        G(c + 1, 1, 1)
        C(0, 0)
        L(c + 4, 0)
        G(c + 2, 2, 0)
        C(1, 1)
        L(c + 5, 1)
        G(c + 3, 3, 1)
        C(2, 0)
        L(c + 6, 2)
        G(c + 4, 0, 0)
        C(3, 1)
        L(c + 7, 3)

    G(NCH - 4, 1, 1)
    C(0, 0)
    G(NCH - 3, 2, 0)
    C(1, 1)
    G(NCH - 2, 3, 1)
    C(2, 0)
    L(NCH - 1, 0)
    G(NCH - 1, 0, 0)
    C(3, 1)
    C(0, 0)

    plsc.subcore_barrier()

    @pl.when(s < 15)
    def _():
        pltpu.sync_copy(spm.at[pl.ds(s * NPT_A, NPT_A)],
                        out_hbm.at[pl.ds(gN + s * NPT_A, NPT_A)])

    @pl.when(s == 15)
    def _():
        pltpu.sync_copy(spm.at[pl.ds(15 * NPT_A, NPT_B)],
                        out_hbm.at[pl.ds(gN + 15 * NPT_A, NPT_B)])


@functools.partial(jax.jit, static_argnums=())
def _sc_edge_pass(ht2, as_t, ad_t, src, dst, z36):
    mesh = plsc.VectorSubcoreMesh(core_axis_name="c", subcore_axis_name="s")
    cp = pltpu.CompilerParams(needs_layout_passes=False,
                              use_tc_tiling_on_sc=False)
    k = pl.kernel(
        _sc_body,
        out_type=jax.ShapeDtypeStruct((G * N, AW), _f32),
        mesh=mesh,
        compiler_params=cp,
        scratch_types=(
            [pltpu.VMEM_SHARED((N, AW), _f32)]
            + [pltpu.VMEM((EK,), jnp.int32) for _ in range(8)]
            + [pltpu.VMEM((EK,), jnp.int32) for _ in range(2)]
            + [pltpu.VMEM((EK, 16), _f32), pltpu.VMEM((EK, 16), _f32),
               pltpu.VMEM((EK, FG), _f32), pltpu.VMEM((EK, AW), _f32)] * 2
            + [pltpu.SemaphoreType.DMA for _ in range(6)]
        ),
    )
    return k(ht2, as_t, ad_t, src, dst, z36)


def _att_mat(a_s, a_d):
    eye = jnp.eye(H, dtype=_f32)
    mk = lambda a: (eye[:, None, :] * a[:, :, None]).reshape(F, H)
    return jnp.concatenate([mk(a_s), mk(a_d)], axis=1)


def kernel(x_node_h, x_global_features, edge_index, batch_idx,
           W1, as1, ad1, b1, W2, as2, ad2, b2, W3, as3, ad3, b3,
           Wm1, bm1, Wm2, bm2, Wm3, bm3, Wm4, bm4):
    src = edge_index[0]
    dst = edge_index[1]
    bidx2 = batch_idx.reshape(N, 1)
    z36 = jnp.zeros((NPT_A, AW), _f32)
    r = jnp.repeat(jnp.eye(4, dtype=_f32), C, axis=1)

    a1 = _att_mat(as1, ad1)
    a2 = _att_mat(as2, ad2)
    a3 = _att_mat(as3, ad3)

    ht, as_t, ad_t = _stage0(x_node_h, bidx2, x_global_features, W1, a1)
    acc = _sc_edge_pass(ht.reshape(G * N, FG), as_t, ad_t, src, dst, z36)

    ht, as_t, ad_t = _mid(acc.reshape(G, N, AW), r, b1.reshape(1, F), W2, a2)
    acc = _sc_edge_pass(ht.reshape(G * N, FG), as_t, ad_t, src, dst, z36)

    ht, as_t, ad_t = _mid(acc.reshape(G, N, AW), r, b2.reshape(1, F), W3, a3)
    acc = _sc_edge_pass(ht.reshape(G * N, FG), as_t, ad_t, src, dst, z36)

    out = _stage3(acc.reshape(G, N, AW), r, b3.reshape(1, F), bidx2,
                  Wm1, bm1.reshape(1, F), Wm2, bm2.reshape(1, F),
                  Wm3, bm3.reshape(1, F), Wm4, bm4.reshape(1, 1))
    return out[:B, 0]


# speedup vs baseline: 152.1145x; 2.8063x over previous
import dataclasses
import functools

import jax
import jax.numpy as jnp
from jax import lax
from jax.experimental import pallas as pl
from jax.experimental.pallas import tpu as pltpu
from jax.experimental.pallas import tpu_sc as plsc

N = 50000
E = 800000
B = 50
H = 4
C = 16
F = 64
G = 2
FG = 32
AW = 36
BN = 2000
NB = N // BN
BPAD = 56
NSUB = 16
EPT = E // NSUB
EK = 80
NCH = EPT // EK
NPT_A = 3128
NPT_B = N - 15 * NPT_A

_f32 = jnp.float32


def _stage0_body(xn_ref, bidx_ref, glob_ref, w_ref, a_ref,
                 ht_ref, as_ref, ad_ref):
    xn = xn_ref[...]
    bidx = bidx_ref[...][:, 0]
    onehot = (bidx[:, None] ==
              lax.broadcasted_iota(jnp.int32, (BN, B), 1)).astype(_f32)
    w = w_ref[...]
    gw = jnp.dot(glob_ref[...], w[32:64, :], preferred_element_type=_f32)
    h = (jnp.dot(xn, w[0:32, :], preferred_element_type=_f32)
         + jnp.dot(onehot, gw, preferred_element_type=_f32))
    a8 = jnp.dot(h, a_ref[...], preferred_element_type=_f32)
    zpad = jnp.zeros((BN, 12), _f32)
    ht_ref[0] = h[:, 0:FG]
    ht_ref[1] = h[:, FG:F]
    as_ref[...] = jnp.concatenate([zpad, a8[:, 0:4]], axis=1)
    ad_ref[...] = jnp.concatenate([zpad, a8[:, 4:8]], axis=1)


def _stage0(xn, bidx2, glob, w1, a1):
    return pl.pallas_call(
        _stage0_body,
        grid=(NB,),
        in_specs=[
            pl.BlockSpec((BN, 32), lambda i: (i, 0)),
            pl.BlockSpec((BN, 1), lambda i: (i, 0)),
            pl.BlockSpec((B, 32), lambda i: (0, 0)),
            pl.BlockSpec((F, F), lambda i: (0, 0)),
            pl.BlockSpec((F, 8), lambda i: (0, 0)),
        ],
        out_specs=[
            pl.BlockSpec((G, BN, FG), lambda i: (0, i, 0)),
            pl.BlockSpec((BN, 16), lambda i: (i, 0)),
            pl.BlockSpec((BN, 16), lambda i: (i, 0)),
        ],
        out_shape=[
            jax.ShapeDtypeStruct((G, N, FG), _f32),
            jax.ShapeDtypeStruct((N, 16), _f32),
            jax.ShapeDtypeStruct((N, 16), _f32),
        ],
    )(xn, bidx2, glob, w1, a1)


def _mid_body(acc_ref, r_ref, b_ref, w_ref, a_ref, ht_ref, as_ref, ad_ref):
    m0 = acc_ref[0]
    m1 = acc_ref[1]
    den = m0[:, FG:AW]
    dd = jnp.dot(den, r_ref[...], preferred_element_type=_f32)
    m = jnp.concatenate([m0[:, 0:FG], m1[:, 0:FG]], axis=1)
    x = m / (dd + 1e-16) + b_ref[...]
    x = jnp.maximum(x, 0.0)
    h = jnp.dot(x, w_ref[...], preferred_element_type=_f32)
    a8 = jnp.dot(h, a_ref[...], preferred_element_type=_f32)
    zpad = jnp.zeros((BN, 12), _f32)
    ht_ref[0] = h[:, 0:FG]
    ht_ref[1] = h[:, FG:F]
    as_ref[...] = jnp.concatenate([zpad, a8[:, 0:4]], axis=1)
    ad_ref[...] = jnp.concatenate([zpad, a8[:, 4:8]], axis=1)


def _mid(acc3, r, b, w, a):
    return pl.pallas_call(
        _mid_body,
        grid=(NB,),
        in_specs=[
            pl.BlockSpec((G, BN, AW), lambda i: (0, i, 0)),
            pl.BlockSpec((4, F), lambda i: (0, 0)),
            pl.BlockSpec((1, F), lambda i: (0, 0)),
            pl.BlockSpec((F, F), lambda i: (0, 0)),
            pl.BlockSpec((F, 8), lambda i: (0, 0)),
        ],
        out_specs=[
            pl.BlockSpec((G, BN, FG), lambda i: (0, i, 0)),
            pl.BlockSpec((BN, 16), lambda i: (i, 0)),
            pl.BlockSpec((BN, 16), lambda i: (i, 0)),
        ],
        out_shape=[
            jax.ShapeDtypeStruct((G, N, FG), _f32),
            jax.ShapeDtypeStruct((N, 16), _f32),
            jax.ShapeDtypeStruct((N, 16), _f32),
        ],
    )(acc3, r, b, w, a)


def _stage3_body(acc_ref, r_ref, b_ref, bidx_ref,
                 wm1_ref, bm1_ref, wm2_ref, bm2_ref,
                 wm3_ref, bm3_ref, wm4_ref, bm4_ref,
                 out_ref, sum_ref, cnt_ref):
    i = pl.program_id(0)

    @pl.when(i == 0)
    def _():
        sum_ref[...] = jnp.zeros((BPAD, F), _f32)
        cnt_ref[...] = jnp.zeros((BPAD, F), _f32)
        out_ref[...] = jnp.zeros((BPAD, 1), _f32)

    m0 = acc_ref[0]
    m1 = acc_ref[1]
    den = m0[:, FG:AW]
    dd = jnp.dot(den, r_ref[...], preferred_element_type=_f32)
    m = jnp.concatenate([m0[:, 0:FG], m1[:, 0:FG]], axis=1)
    x = m / (dd + 1e-16) + b_ref[...]

    bidx = bidx_ref[...][:, 0]
    onehot = (bidx[:, None] ==
              lax.broadcasted_iota(jnp.int32, (BN, BPAD), 1)).astype(_f32)
    dn = (((0,), (0,)), ((), ()))
    sum_ref[...] += lax.dot_general(onehot, x, dn,
                                    preferred_element_type=_f32)
    cnt_ref[...] += lax.dot_general(onehot, jnp.ones((BN, F), _f32), dn,
                                    preferred_element_type=_f32)

    @pl.when(i == NB - 1)
    def _():
        g = sum_ref[...] / jnp.maximum(cnt_ref[...], 1.0)
        h1 = jnp.maximum(jnp.dot(g, wm1_ref[...],
                                 preferred_element_type=_f32)
                         + bm1_ref[...], 0.0)
        h2 = jnp.maximum(jnp.dot(h1, wm2_ref[...],
                                 preferred_element_type=_f32)
                         + bm2_ref[...], 0.0)
        h3 = jnp.maximum(jnp.dot(h2, wm3_ref[...],
                                 preferred_element_type=_f32)
                         + bm3_ref[...], 0.0)
        out_ref[...] = (jnp.dot(h3, wm4_ref[...],
                                preferred_element_type=_f32)
                        + bm4_ref[...])


def _stage3(acc3, r, b, bidx2, wm1, bm1, wm2, bm2, wm3, bm3, wm4, bm4):
    full = lambda s: pl.BlockSpec(s, lambda i: tuple(0 for _ in s))
    return pl.pallas_call(
        _stage3_body,
        grid=(NB,),
        in_specs=[
            pl.BlockSpec((G, BN, AW), lambda i: (0, i, 0)),
            full((4, F)),
            full((1, F)),
            pl.BlockSpec((BN, 1), lambda i: (i, 0)),
            full((F, F)), full((1, F)),
            full((F, F)), full((1, F)),
            full((F, F)), full((1, F)),
            full((F, 1)), full((1, 1)),
        ],
        out_specs=pl.BlockSpec((BPAD, 1), lambda i: (0, 0)),
        out_shape=jax.ShapeDtypeStruct((BPAD, 1), _f32),
        scratch_shapes=[
            pltpu.VMEM((BPAD, F), _f32),
            pltpu.VMEM((BPAD, F), _f32),
        ],
    )(acc3, r, b, bidx2, wm1, bm1, wm2, bm2, wm3, bm3, wm4, bm4)


def _sc_body(ht_hbm, as_hbm, ad_hbm, src_hbm, dst_hbm, z_hbm,
             out_hbm,
             spm,
             sv0, dv0, sv1, dv1, sv2, dv2, sv3, dv3,
             hx0, hx1, sdv0, sdv1,
             ab0, db0, hb0, mb0,
             ab1, db1, hb1, mb1,
             si0, si1, si2, si3, sg0, sg1, ss0, ss1):
    g = lax.axis_index("c")
    s = lax.axis_index("s")
    gN = g * N
    lane = lax.iota(jnp.int32, 16)
    m0v = (lane == 12 + 2 * g).astype(_f32)
    m1v = (lane == 13 + 2 * g).astype(_f32)

    @pl.when(s < 15)
    def _():
        pltpu.sync_copy(z_hbm, spm.at[pl.ds(s * NPT_A, NPT_A)])

    @pl.when(s == 15)
    def _():
        pltpu.sync_copy(z_hbm.at[pl.ds(0, NPT_B)],
                        spm.at[pl.ds(15 * NPT_A, NPT_B)])

    plsc.subcore_barrier()

    sv = (sv0, sv1, sv2, sv3)
    dv = (dv0, dv1, dv2, dv3)
    si = (si0, si1, si2, si3)
    hx = (hx0, hx1)
    sdv = (sdv0, sdv1)
    ab = (ab0, ab1)
    db = (db0, db1)
    hb = (hb0, hb1)
    mb = (mb0, mb1)
    sg = (sg0, sg1)
    ss = (ss0, ss1)

    def idx_pair(i, r):
        off = s * EPT + i * EK
        return (pltpu.make_async_copy(src_hbm.at[pl.ds(off, EK)], sv[r], si[r]),
                pltpu.make_async_copy(dst_hbm.at[pl.ds(off, EK)], dv[r], si[r]))

    def L(i, r):
        c1, c2 = idx_pair(i, r)
        c1.start()
        c2.start()

    def gath(r, b):
        return (pltpu.make_async_copy(as_hbm.at[sv[r]], ab[b], sg[b]),
                pltpu.make_async_copy(ad_hbm.at[dv[r]], db[b], sg[b]),
                pltpu.make_async_copy(ht_hbm.at[hx[b]], hb[b], sg[b]))

    def G(i, r, b):
        c1, c2 = idx_pair(i, r)
        c1.wait()
        c2.wait()

        @pl.loop(0, EK, step=16)
        def _(j):
            hx[b][pl.ds(j, 16)] = sv[r][pl.ds(j, 16)] + gN

        g1, g2, g3 = gath(r, b)
        g1.start()
        g2.start()
        g3.start()

    def compute(r, b):
        @pl.loop(0, EK, step=16)
        def _(j):
            sdv[b][pl.ds(j, 16)] = dv[r][pl.ds(j, 16)]

        @plsc.parallel_loop(0, EK, 1, unroll=2)
        def _(e):
            t = ab[b][e, 0:16] + db[b][e, 0:16]
            ex = jnp.exp(jnp.maximum(t, 0.2 * t))
            mb[b][e, 20:36] = ex
            e0 = jnp.sum(ex * m0v)
            e1 = jnp.sum(ex * m1v)
            mb[b][e, 0:16] = hb[b][e, 0:16] * e0
            mb[b][e, 16:32] = hb[b][e, 16:32] * e1

    def C0(r, b):
        g1, g2, g3 = gath(r, b)
        g1.wait()
        g2.wait()
        g3.wait()
        compute(r, b)
        pltpu.async_copy(mb[b], spm.at[sdv[b]], ss[b], add=True)

    def C(r, b):
        g1, g2, g3 = gath(r, b)
        g1.wait()
        g2.wait()
        g3.wait()
        pltpu.make_async_copy(mb[b], spm.at[sdv[b]], ss[b]).wait()
        compute(r, b)
        pltpu.async_copy(mb[b], spm.at[sdv[b]], ss[b], add=True)

    L(0, 0)
    L(1, 1)
    L(2, 2)
    L(3, 3)
    G(0, 0, 0)
    G(1, 1, 1)
    C0(0, 0)
    L(4, 0)
    G(2, 2, 0)
    C0(1, 1)
    L(5, 1)
    G(3, 3, 1)

    @pl.loop(0, 154)
    def _(k):
        c = 4 * k + 2
        C(2, 0)
        L(c + 4, 2)
        G(c + 2, 0, 0)
        C(3, 1)
        L(c + 5, 3)
        G(c + 3, 1, 1)
        C(0, 0)
        L(c + 6, 0)
        G(c + 4, 2, 0)
        C(1, 1)
        L(c + 7, 1)
        G(c + 5, 3, 1)

    C(2, 0)
    L(622, 2)
    G(620, 0, 0)
    C(3, 1)
    L(623, 3)
    G(621, 1, 1)
    C(0, 0)
    L(624, 0)
    G(622, 2, 0)
    C(1, 1)
    G(623, 3, 1)
    C(2, 0)
    G(624, 0, 0)
    C(3, 1)
    C(0, 0)
    pltpu.make_async_copy(mb[0], spm.at[sdv[0]], ss[0]).wait()
    pltpu.make_async_copy(mb[1], spm.at[sdv[1]], ss[1]).wait()

    plsc.subcore_barrier()

    @pl.when(s < 15)
    def _():
        pltpu.sync_copy(spm.at[pl.ds(s * NPT_A, NPT_A)],
                        out_hbm.at[pl.ds(gN + s * NPT_A, NPT_A)])

    @pl.when(s == 15)
    def _():
        pltpu.sync_copy(spm.at[pl.ds(15 * NPT_A, NPT_B)],
                        out_hbm.at[pl.ds(gN + 15 * NPT_A, NPT_B)])


@functools.partial(jax.jit, static_argnums=())
def _sc_edge_pass(ht2, as_t, ad_t, src, dst, z36):
    mesh = plsc.VectorSubcoreMesh(core_axis_name="c", subcore_axis_name="s")
    cp = pltpu.CompilerParams(needs_layout_passes=False,
                              use_tc_tiling_on_sc=False)
    k = pl.kernel(
        _sc_body,
        out_type=jax.ShapeDtypeStruct((G * N, AW), _f32),
        mesh=mesh,
        compiler_params=cp,
        scratch_types=(
            [pltpu.VMEM_SHARED((N, AW), _f32)]
            + [pltpu.VMEM((EK,), jnp.int32) for _ in range(8)]
            + [pltpu.VMEM((EK,), jnp.int32) for _ in range(4)]
            + [pltpu.VMEM((EK, 16), _f32), pltpu.VMEM((EK, 16), _f32),
               pltpu.VMEM((EK, FG), _f32), pltpu.VMEM((EK, AW), _f32)] * 2
            + [pltpu.SemaphoreType.DMA for _ in range(8)]
        ),
    )
    return k(ht2, as_t, ad_t, src, dst, z36)


def _att_mat(a_s, a_d):
    eye = jnp.eye(H, dtype=_f32)
    mk = lambda a: (eye[:, None, :] * a[:, :, None]).reshape(F, H)
    return jnp.concatenate([mk(a_s), mk(a_d)], axis=1)


def kernel(x_node_h, x_global_features, edge_index, batch_idx,
           W1, as1, ad1, b1, W2, as2, ad2, b2, W3, as3, ad3, b3,
           Wm1, bm1, Wm2, bm2, Wm3, bm3, Wm4, bm4):
    src = edge_index[0]
    dst = edge_index[1]
    bidx2 = batch_idx.reshape(N, 1)
    z36 = jnp.zeros((NPT_A, AW), _f32)
    r = jnp.repeat(jnp.eye(4, dtype=_f32), C, axis=1)

    a1 = _att_mat(as1, ad1)
    a2 = _att_mat(as2, ad2)
    a3 = _att_mat(as3, ad3)

    ht, as_t, ad_t = _stage0(x_node_h, bidx2, x_global_features, W1, a1)
    acc = _sc_edge_pass(ht.reshape(G * N, FG), as_t, ad_t, src, dst, z36)

    ht, as_t, ad_t = _mid(acc.reshape(G, N, AW), r, b1.reshape(1, F), W2, a2)
    acc = _sc_edge_pass(ht.reshape(G * N, FG), as_t, ad_t, src, dst, z36)

    ht, as_t, ad_t = _mid(acc.reshape(G, N, AW), r, b2.reshape(1, F), W3, a3)
    acc = _sc_edge_pass(ht.reshape(G * N, FG), as_t, ad_t, src, dst, z36)

    out = _stage3(acc.reshape(G, N, AW), r, b3.reshape(1, F), bidx2,
                  Wm1, bm1.reshape(1, F), Wm2, bm2.reshape(1, F),
                  Wm3, bm3.reshape(1, F), Wm4, bm4.reshape(1, 1))
    return out[:B, 0]


# speedup vs baseline: 165.1979x; 1.0860x over previous
import dataclasses
import functools

import jax
import jax.numpy as jnp
from jax import lax
from jax.experimental import pallas as pl
from jax.experimental.pallas import tpu as pltpu
from jax.experimental.pallas import tpu_sc as plsc

N = 50000
E = 800000
B = 50
H = 4
C = 16
F = 64
G = 2
FG = 32
AW = 36
BN = 2000
NB = N // BN
BPAD = 56
HW = 48
NSUB = 16
EPT = E // NSUB
EK = 80
NCH = EPT // EK
NPT_A = 3128
NPT_B = N - 15 * NPT_A

_f32 = jnp.float32


def _stage0_body(xn_ref, bidx_ref, glob_ref, w_ref, a_ref,
                 ht_ref, ad_ref):
    xn = xn_ref[...]
    bidx = bidx_ref[...][:, 0]
    onehot = (bidx[:, None] ==
              lax.broadcasted_iota(jnp.int32, (BN, B), 1)).astype(_f32)
    w = w_ref[...]
    gw = jnp.dot(glob_ref[...], w[32:64, :], preferred_element_type=_f32)
    h = (jnp.dot(xn, w[0:32, :], preferred_element_type=_f32)
         + jnp.dot(onehot, gw, preferred_element_type=_f32))
    a8 = jnp.dot(h, a_ref[...], preferred_element_type=_f32)
    zpad = jnp.zeros((BN, 12), _f32)
    ht_ref[0] = jnp.concatenate([h[:, 0:FG], zpad, a8[:, 0:4]], axis=1)
    ht_ref[1] = jnp.concatenate([h[:, FG:F], zpad, a8[:, 0:4]], axis=1)
    ad_ref[...] = jnp.concatenate([zpad, a8[:, 4:8]], axis=1)


def _stage0(xn, bidx2, glob, w1, a1):
    return pl.pallas_call(
        _stage0_body,
        grid=(NB,),
        in_specs=[
            pl.BlockSpec((BN, 32), lambda i: (i, 0)),
            pl.BlockSpec((BN, 1), lambda i: (i, 0)),
            pl.BlockSpec((B, 32), lambda i: (0, 0)),
            pl.BlockSpec((F, F), lambda i: (0, 0)),
            pl.BlockSpec((F, 8), lambda i: (0, 0)),
        ],
        out_specs=[
            pl.BlockSpec((G, BN, HW), lambda i: (0, i, 0)),
            pl.BlockSpec((BN, 16), lambda i: (i, 0)),
        ],
        out_shape=[
            jax.ShapeDtypeStruct((G, N, HW), _f32),
            jax.ShapeDtypeStruct((N, 16), _f32),
        ],
    )(xn, bidx2, glob, w1, a1)


def _mid_body(acc_ref, r_ref, b_ref, w_ref, a_ref, ht_ref, ad_ref):
    m0 = acc_ref[0]
    m1 = acc_ref[1]
    den = m0[:, FG:AW]
    dd = jnp.dot(den, r_ref[...], preferred_element_type=_f32)
    m = jnp.concatenate([m0[:, 0:FG], m1[:, 0:FG]], axis=1)
    x = m / (dd + 1e-16) + b_ref[...]
    x = jnp.maximum(x, 0.0)
    h = jnp.dot(x, w_ref[...], preferred_element_type=_f32)
    a8 = jnp.dot(h, a_ref[...], preferred_element_type=_f32)
    zpad = jnp.zeros((BN, 12), _f32)
    ht_ref[0] = jnp.concatenate([h[:, 0:FG], zpad, a8[:, 0:4]], axis=1)
    ht_ref[1] = jnp.concatenate([h[:, FG:F], zpad, a8[:, 0:4]], axis=1)
    ad_ref[...] = jnp.concatenate([zpad, a8[:, 4:8]], axis=1)


def _mid(acc3, r, b, w, a):
    return pl.pallas_call(
        _mid_body,
        grid=(NB,),
        in_specs=[
            pl.BlockSpec((G, BN, AW), lambda i: (0, i, 0)),
            pl.BlockSpec((4, F), lambda i: (0, 0)),
            pl.BlockSpec((1, F), lambda i: (0, 0)),
            pl.BlockSpec((F, F), lambda i: (0, 0)),
            pl.BlockSpec((F, 8), lambda i: (0, 0)),
        ],
        out_specs=[
            pl.BlockSpec((G, BN, HW), lambda i: (0, i, 0)),
            pl.BlockSpec((BN, 16), lambda i: (i, 0)),
        ],
        out_shape=[
            jax.ShapeDtypeStruct((G, N, HW), _f32),
            jax.ShapeDtypeStruct((N, 16), _f32),
        ],
    )(acc3, r, b, w, a)


def _stage3_body(acc_ref, r_ref, b_ref, bidx_ref,
                 wm1_ref, bm1_ref, wm2_ref, bm2_ref,
                 wm3_ref, bm3_ref, wm4_ref, bm4_ref,
                 out_ref, sum_ref, cnt_ref):
    i = pl.program_id(0)

    @pl.when(i == 0)
    def _():
        sum_ref[...] = jnp.zeros((BPAD, F), _f32)
        cnt_ref[...] = jnp.zeros((BPAD, F), _f32)
        out_ref[...] = jnp.zeros((BPAD, 1), _f32)

    m0 = acc_ref[0]
    m1 = acc_ref[1]
    den = m0[:, FG:AW]
    dd = jnp.dot(den, r_ref[...], preferred_element_type=_f32)
    m = jnp.concatenate([m0[:, 0:FG], m1[:, 0:FG]], axis=1)
    x = m / (dd + 1e-16) + b_ref[...]

    bidx = bidx_ref[...][:, 0]
    onehot = (bidx[:, None] ==
              lax.broadcasted_iota(jnp.int32, (BN, BPAD), 1)).astype(_f32)
    dn = (((0,), (0,)), ((), ()))
    sum_ref[...] += lax.dot_general(onehot, x, dn,
                                    preferred_element_type=_f32)
    cnt_ref[...] += lax.dot_general(onehot, jnp.ones((BN, F), _f32), dn,
                                    preferred_element_type=_f32)

    @pl.when(i == NB - 1)
    def _():
        g = sum_ref[...] / jnp.maximum(cnt_ref[...], 1.0)
        h1 = jnp.maximum(jnp.dot(g, wm1_ref[...],
                                 preferred_element_type=_f32)
                         + bm1_ref[...], 0.0)
        h2 = jnp.maximum(jnp.dot(h1, wm2_ref[...],
                                 preferred_element_type=_f32)
                         + bm2_ref[...], 0.0)
        h3 = jnp.maximum(jnp.dot(h2, wm3_ref[...],
                                 preferred_element_type=_f32)
                         + bm3_ref[...], 0.0)
        out_ref[...] = (jnp.dot(h3, wm4_ref[...],
                                preferred_element_type=_f32)
                        + bm4_ref[...])


def _stage3(acc3, r, b, bidx2, wm1, bm1, wm2, bm2, wm3, bm3, wm4, bm4):
    full = lambda s: pl.BlockSpec(s, lambda i: tuple(0 for _ in s))
    return pl.pallas_call(
        _stage3_body,
        grid=(NB,),
        in_specs=[
            pl.BlockSpec((G, BN, AW), lambda i: (0, i, 0)),
            full((4, F)),
            full((1, F)),
            pl.BlockSpec((BN, 1), lambda i: (i, 0)),
            full((F, F)), full((1, F)),
            full((F, F)), full((1, F)),
            full((F, F)), full((1, F)),
            full((F, 1)), full((1, 1)),
        ],
        out_specs=pl.BlockSpec((BPAD, 1), lambda i: (0, 0)),
        out_shape=jax.ShapeDtypeStruct((BPAD, 1), _f32),
        scratch_shapes=[
            pltpu.VMEM((BPAD, F), _f32),
            pltpu.VMEM((BPAD, F), _f32),
        ],
    )(acc3, r, b, bidx2, wm1, bm1, wm2, bm2, wm3, bm3, wm4, bm4)


def _sc_body(ht_hbm, ad_hbm, src_hbm, dst_hbm, z_hbm,
             out_hbm,
             spm,
             sv0, dv0, sv1, dv1, sv2, dv2, sv3, dv3,
             hx0, hx1, sdv0, sdv1,
             db0, hb0, mb0,
             db1, hb1, mb1,
             si0, si1, si2, si3, sg0, sg1, ss0, ss1):
    g = lax.axis_index("c")
    s = lax.axis_index("s")
    gN = g * N
    i0v = jnp.full((16,), 12 + 2 * g, jnp.int32)
    i1v = i0v + 1
    _gdn = lax.GatherDimensionNumbers(
        offset_dims=(), collapsed_slice_dims=(0,), start_index_map=(0,))

    def _bcast(vec, idx):
        return lax.gather(vec, idx[:, None], _gdn, (1,),
                          mode=lax.GatherScatterMode.PROMISE_IN_BOUNDS)

    @pl.when(s < 15)
    def _():
        pltpu.sync_copy(z_hbm, spm.at[pl.ds(s * NPT_A, NPT_A)])

    @pl.when(s == 15)
    def _():
        pltpu.sync_copy(z_hbm.at[pl.ds(0, NPT_B)],
                        spm.at[pl.ds(15 * NPT_A, NPT_B)])

    plsc.subcore_barrier()

    sv = (sv0, sv1, sv2, sv3)
    dv = (dv0, dv1, dv2, dv3)
    si = (si0, si1, si2, si3)
    hx = (hx0, hx1)
    sdv = (sdv0, sdv1)
    db = (db0, db1)
    hb = (hb0, hb1)
    mb = (mb0, mb1)
    sg = (sg0, sg1)
    ss = (ss0, ss1)

    def idx_pair(i, r):
        off = s * EPT + i * EK
        return (pltpu.make_async_copy(src_hbm.at[pl.ds(off, EK)], sv[r], si[r]),
                pltpu.make_async_copy(dst_hbm.at[pl.ds(off, EK)], dv[r], si[r]))

    def L(i, r):
        c1, c2 = idx_pair(i, r)
        c1.start()
        c2.start()

    def gath(r, b):
        return (pltpu.make_async_copy(ad_hbm.at[dv[r]], db[b], sg[b]),
                pltpu.make_async_copy(ht_hbm.at[hx[b]], hb[b], sg[b]))

    def G(i, r, b):
        c1, c2 = idx_pair(i, r)
        c1.wait()
        c2.wait()

        @pl.loop(0, EK, step=16)
        def _(j):
            hx[b][pl.ds(j, 16)] = sv[r][pl.ds(j, 16)] + gN

        g1, g2 = gath(r, b)
        g1.start()
        g2.start()

    def compute(r, b):
        @pl.loop(0, EK, step=16)
        def _(j):
            sdv[b][pl.ds(j, 16)] = dv[r][pl.ds(j, 16)]

        @plsc.parallel_loop(0, EK, 1, unroll=4)
        def _(e):
            t = hb[b][e, FG:HW] + db[b][e, 0:16]
            ex = jnp.exp(jnp.maximum(t, 0.2 * t))
            mb[b][e, 20:36] = ex
            mb[b][e, 0:16] = hb[b][e, 0:16] * _bcast(ex, i0v)
            mb[b][e, 16:32] = hb[b][e, 16:32] * _bcast(ex, i1v)

    def C0(r, b):
        g1, g2 = gath(r, b)
        g1.wait()
        g2.wait()
        compute(r, b)
        pltpu.async_copy(mb[b], spm.at[sdv[b]], ss[b], add=True)

    def C(r, b):
        g1, g2 = gath(r, b)
        g1.wait()
        g2.wait()
        pltpu.make_async_copy(mb[b], spm.at[sdv[b]], ss[b]).wait()
        compute(r, b)
        pltpu.async_copy(mb[b], spm.at[sdv[b]], ss[b], add=True)

    L(0, 0)
    L(1, 1)
    L(2, 2)
    L(3, 3)
    G(0, 0, 0)
    G(1, 1, 1)
    C0(0, 0)
    L(4, 0)
    G(2, 2, 0)
    C0(1, 1)
    L(5, 1)
    G(3, 3, 1)

    @pl.loop(0, 154)
    def _(k):
        c = 4 * k + 2
        C(2, 0)
        L(c + 4, 2)
        G(c + 2, 0, 0)
        C(3, 1)
        L(c + 5, 3)
        G(c + 3, 1, 1)
        C(0, 0)
        L(c + 6, 0)
        G(c + 4, 2, 0)
        C(1, 1)
        L(c + 7, 1)
        G(c + 5, 3, 1)

    C(2, 0)
    L(622, 2)
    G(620, 0, 0)
    C(3, 1)
    L(623, 3)
    G(621, 1, 1)
    C(0, 0)
    L(624, 0)
    G(622, 2, 0)
    C(1, 1)
    G(623, 3, 1)
    C(2, 0)
    G(624, 0, 0)
    C(3, 1)
    C(0, 0)
    pltpu.make_async_copy(mb[0], spm.at[sdv[0]], ss[0]).wait()
    pltpu.make_async_copy(mb[1], spm.at[sdv[1]], ss[1]).wait()

    plsc.subcore_barrier()

    @pl.when(s < 15)
    def _():
        pltpu.sync_copy(spm.at[pl.ds(s * NPT_A, NPT_A)],
                        out_hbm.at[pl.ds(gN + s * NPT_A, NPT_A)])

    @pl.when(s == 15)
    def _():
        pltpu.sync_copy(spm.at[pl.ds(15 * NPT_A, NPT_B)],
                        out_hbm.at[pl.ds(gN + 15 * NPT_A, NPT_B)])


@functools.partial(jax.jit, static_argnums=())
def _sc_edge_pass(ht2, ad_t, src, dst, z36):
    mesh = plsc.VectorSubcoreMesh(core_axis_name="c", subcore_axis_name="s")
    cp = pltpu.CompilerParams(needs_layout_passes=False,
                              use_tc_tiling_on_sc=False)
    k = pl.kernel(
        _sc_body,
        out_type=jax.ShapeDtypeStruct((G * N, AW), _f32),
        mesh=mesh,
        compiler_params=cp,
        scratch_types=(
            [pltpu.VMEM_SHARED((N, AW), _f32)]
            + [pltpu.VMEM((EK,), jnp.int32) for _ in range(8)]
            + [pltpu.VMEM((EK,), jnp.int32) for _ in range(4)]
            + [pltpu.VMEM((EK, 16), _f32),
               pltpu.VMEM((EK, HW), _f32), pltpu.VMEM((EK, AW), _f32)] * 2
            + [pltpu.SemaphoreType.DMA for _ in range(8)]
        ),
    )
    return k(ht2, ad_t, src, dst, z36)


def _att_mat(a_s, a_d):
    eye = jnp.eye(H, dtype=_f32)
    mk = lambda a: (eye[:, None, :] * a[:, :, None]).reshape(F, H)
    return jnp.concatenate([mk(a_s), mk(a_d)], axis=1)


def kernel(x_node_h, x_global_features, edge_index, batch_idx,
           W1, as1, ad1, b1, W2, as2, ad2, b2, W3, as3, ad3, b3,
           Wm1, bm1, Wm2, bm2, Wm3, bm3, Wm4, bm4):
    src = edge_index[0]
    dst = edge_index[1]
    bidx2 = batch_idx.reshape(N, 1)
    z36 = jnp.zeros((NPT_A, AW), _f32)
    r = jnp.repeat(jnp.eye(4, dtype=_f32), C, axis=1)

    a1 = _att_mat(as1, ad1)
    a2 = _att_mat(as2, ad2)
    a3 = _att_mat(as3, ad3)

    ht, ad_t = _stage0(x_node_h, bidx2, x_global_features, W1, a1)
    acc = _sc_edge_pass(ht.reshape(G * N, HW), ad_t, src, dst, z36)

    ht, ad_t = _mid(acc.reshape(G, N, AW), r, b1.reshape(1, F), W2, a2)
    acc = _sc_edge_pass(ht.reshape(G * N, HW), ad_t, src, dst, z36)

    ht, ad_t = _mid(acc.reshape(G, N, AW), r, b2.reshape(1, F), W3, a3)
    acc = _sc_edge_pass(ht.reshape(G * N, HW), ad_t, src, dst, z36)

    out = _stage3(acc.reshape(G, N, AW), r, b3.reshape(1, F), bidx2,
                  Wm1, bm1.reshape(1, F), Wm2, bm2.reshape(1, F),
                  Wm3, bm3.reshape(1, F), Wm4, bm4.reshape(1, 1))
    return out[:B, 0]


# speedup vs baseline: 165.5582x; 1.0022x over previous
import dataclasses
import functools

import jax
import jax.numpy as jnp
from jax import lax
from jax.experimental import pallas as pl
from jax.experimental.pallas import tpu as pltpu
from jax.experimental.pallas import tpu_sc as plsc

N = 50000
E = 800000
B = 50
H = 4
C = 16
F = 64
G = 2
FG = 32
AW = 36
BN = 2000
NB = N // BN
BPAD = 56
HW = 48
NSUB = 16
EPT = E // NSUB
EK = 80
NCH = EPT // EK
NPT_A = 3128
NPT_B = N - 15 * NPT_A

_f32 = jnp.float32


def _stage0_body(xn_ref, bidx_ref, glob_ref, w_ref, a_ref,
                 ht_ref, ad_ref):
    xn = xn_ref[...]
    bidx = bidx_ref[...][:, 0]
    onehot = (bidx[:, None] ==
              lax.broadcasted_iota(jnp.int32, (BN, B), 1)).astype(_f32)
    w = w_ref[...]
    gw = jnp.dot(glob_ref[...], w[32:64, :], preferred_element_type=_f32)
    h = (jnp.dot(xn, w[0:32, :], preferred_element_type=_f32)
         + jnp.dot(onehot, gw, preferred_element_type=_f32))
    a8 = jnp.dot(h, a_ref[...], preferred_element_type=_f32)
    zpad = jnp.zeros((BN, 12), _f32)
    ht_ref[0] = jnp.concatenate([h[:, 0:FG], zpad, a8[:, 0:4]], axis=1)
    ht_ref[1] = jnp.concatenate([h[:, FG:F], zpad, a8[:, 0:4]], axis=1)
    ad_ref[...] = jnp.concatenate([zpad, a8[:, 4:8]], axis=1)


def _stage0(xn, bidx2, glob, w1, a1):
    return pl.pallas_call(
        _stage0_body,
        grid=(NB,),
        in_specs=[
            pl.BlockSpec((BN, 32), lambda i: (i, 0)),
            pl.BlockSpec((BN, 1), lambda i: (i, 0)),
            pl.BlockSpec((B, 32), lambda i: (0, 0)),
            pl.BlockSpec((F, F), lambda i: (0, 0)),
            pl.BlockSpec((F, 8), lambda i: (0, 0)),
        ],
        out_specs=[
            pl.BlockSpec((G, BN, HW), lambda i: (0, i, 0)),
            pl.BlockSpec((BN, 16), lambda i: (i, 0)),
        ],
        out_shape=[
            jax.ShapeDtypeStruct((G, N, HW), _f32),
            jax.ShapeDtypeStruct((N, 16), _f32),
        ],
    )(xn, bidx2, glob, w1, a1)


def _mid_body(acc_ref, r_ref, b_ref, w_ref, a_ref, ht_ref, ad_ref):
    m0 = acc_ref[0]
    m1 = acc_ref[1]
    den = m0[:, FG:AW]
    dd = jnp.dot(den, r_ref[...], preferred_element_type=_f32)
    m = jnp.concatenate([m0[:, 0:FG], m1[:, 0:FG]], axis=1)
    x = m / (dd + 1e-16) + b_ref[...]
    x = jnp.maximum(x, 0.0)
    h = jnp.dot(x, w_ref[...], preferred_element_type=_f32)
    a8 = jnp.dot(h, a_ref[...], preferred_element_type=_f32)
    zpad = jnp.zeros((BN, 12), _f32)
    ht_ref[0] = jnp.concatenate([h[:, 0:FG], zpad, a8[:, 0:4]], axis=1)
    ht_ref[1] = jnp.concatenate([h[:, FG:F], zpad, a8[:, 0:4]], axis=1)
    ad_ref[...] = jnp.concatenate([zpad, a8[:, 4:8]], axis=1)


def _mid(acc3, r, b, w, a):
    return pl.pallas_call(
        _mid_body,
        grid=(NB,),
        in_specs=[
            pl.BlockSpec((G, BN, AW), lambda i: (0, i, 0)),
            pl.BlockSpec((4, F), lambda i: (0, 0)),
            pl.BlockSpec((1, F), lambda i: (0, 0)),
            pl.BlockSpec((F, F), lambda i: (0, 0)),
            pl.BlockSpec((F, 8), lambda i: (0, 0)),
        ],
        out_specs=[
            pl.BlockSpec((G, BN, HW), lambda i: (0, i, 0)),
            pl.BlockSpec((BN, 16), lambda i: (i, 0)),
        ],
        out_shape=[
            jax.ShapeDtypeStruct((G, N, HW), _f32),
            jax.ShapeDtypeStruct((N, 16), _f32),
        ],
    )(acc3, r, b, w, a)


def _stage3_body(acc_ref, r_ref, b_ref, bidx_ref,
                 wm1_ref, bm1_ref, wm2_ref, bm2_ref,
                 wm3_ref, bm3_ref, wm4_ref, bm4_ref,
                 out_ref, sum_ref, cnt_ref):
    i = pl.program_id(0)

    @pl.when(i == 0)
    def _():
        sum_ref[...] = jnp.zeros((BPAD, F), _f32)
        cnt_ref[...] = jnp.zeros((BPAD, F), _f32)
        out_ref[...] = jnp.zeros((BPAD, 1), _f32)

    m0 = acc_ref[0]
    m1 = acc_ref[1]
    den = m0[:, FG:AW]
    dd = jnp.dot(den, r_ref[...], preferred_element_type=_f32)
    m = jnp.concatenate([m0[:, 0:FG], m1[:, 0:FG]], axis=1)
    x = m / (dd + 1e-16) + b_ref[...]

    bidx = bidx_ref[...][:, 0]
    onehot = (bidx[:, None] ==
              lax.broadcasted_iota(jnp.int32, (BN, BPAD), 1)).astype(_f32)
    dn = (((0,), (0,)), ((), ()))
    sum_ref[...] += lax.dot_general(onehot, x, dn,
                                    preferred_element_type=_f32)
    cnt_ref[...] += lax.dot_general(onehot, jnp.ones((BN, F), _f32), dn,
                                    preferred_element_type=_f32)

    @pl.when(i == NB - 1)
    def _():
        g = sum_ref[...] / jnp.maximum(cnt_ref[...], 1.0)
        h1 = jnp.maximum(jnp.dot(g, wm1_ref[...],
                                 preferred_element_type=_f32)
                         + bm1_ref[...], 0.0)
        h2 = jnp.maximum(jnp.dot(h1, wm2_ref[...],
                                 preferred_element_type=_f32)
                         + bm2_ref[...], 0.0)
        h3 = jnp.maximum(jnp.dot(h2, wm3_ref[...],
                                 preferred_element_type=_f32)
                         + bm3_ref[...], 0.0)
        out_ref[...] = (jnp.dot(h3, wm4_ref[...],
                                preferred_element_type=_f32)
                        + bm4_ref[...])


def _stage3(acc3, r, b, bidx2, wm1, bm1, wm2, bm2, wm3, bm3, wm4, bm4):
    full = lambda s: pl.BlockSpec(s, lambda i: tuple(0 for _ in s))
    return pl.pallas_call(
        _stage3_body,
        grid=(NB,),
        in_specs=[
            pl.BlockSpec((G, BN, AW), lambda i: (0, i, 0)),
            full((4, F)),
            full((1, F)),
            pl.BlockSpec((BN, 1), lambda i: (i, 0)),
            full((F, F)), full((1, F)),
            full((F, F)), full((1, F)),
            full((F, F)), full((1, F)),
            full((F, 1)), full((1, 1)),
        ],
        out_specs=pl.BlockSpec((BPAD, 1), lambda i: (0, 0)),
        out_shape=jax.ShapeDtypeStruct((BPAD, 1), _f32),
        scratch_shapes=[
            pltpu.VMEM((BPAD, F), _f32),
            pltpu.VMEM((BPAD, F), _f32),
        ],
    )(acc3, r, b, bidx2, wm1, bm1, wm2, bm2, wm3, bm3, wm4, bm4)


def _sc_body(ht_hbm, ad_hbm, src_hbm, dst_hbm, z_hbm,
             out_hbm,
             spm,
             sv0, dv0, sv1, dv1, sv2, dv2, sv3, dv3,
             hx0, hx1, sdv0, sdv1,
             db0, hb0, mb0,
             db1, hb1, mb1,
             si0, si1, si2, si3, sg0, sg1, ss0, ss1):
    g = lax.axis_index("c")
    s = lax.axis_index("s")
    gN = g * N
    i0v = jnp.full((16,), 12 + 2 * g, jnp.int32)
    i1v = i0v + 1
    _gdn = lax.GatherDimensionNumbers(
        offset_dims=(), collapsed_slice_dims=(0,), start_index_map=(0,))

    def _bcast(vec, idx):
        return lax.gather(vec, idx[:, None], _gdn, (1,),
                          mode=lax.GatherScatterMode.PROMISE_IN_BOUNDS)

    @pl.when(s < 15)
    def _():
        pltpu.sync_copy(z_hbm, spm.at[pl.ds(s * NPT_A, NPT_A)])

    @pl.when(s == 15)
    def _():
        pltpu.sync_copy(z_hbm.at[pl.ds(0, NPT_B)],
                        spm.at[pl.ds(15 * NPT_A, NPT_B)])

    plsc.subcore_barrier()

    sv = (sv0, sv1, sv2, sv3)
    dv = (dv0, dv1, dv2, dv3)
    si = (si0, si1, si2, si3)
    hx = (hx0, hx1)
    sdv = (sdv0, sdv1)
    db = (db0, db1)
    hb = (hb0, hb1)
    mb = (mb0, mb1)
    sg = (sg0, sg1)
    ss = (ss0, ss1)

    def idx_pair(i, r):
        off = s * EPT + i * EK
        return (pltpu.make_async_copy(src_hbm.at[pl.ds(off, EK)], sv[r], si[r]),
                pltpu.make_async_copy(dst_hbm.at[pl.ds(off, EK)], dv[r], si[r]))

    def L(i, r):
        c1, c2 = idx_pair(i, r)
        c1.start()
        c2.start()

    def gath(r, b):
        return (pltpu.make_async_copy(ad_hbm.at[dv[r]], db[b], sg[b]),
                pltpu.make_async_copy(ht_hbm.at[hx[b]], hb[b], sg[b]))

    def G(i, r, b):
        c1, c2 = idx_pair(i, r)
        c1.wait()
        c2.wait()

        @pl.loop(0, EK, step=16)
        def _(j):
            hx[b][pl.ds(j, 16)] = sv[r][pl.ds(j, 16)] + gN

        g1, g2 = gath(r, b)
        g1.start()
        g2.start()

    def compute(r, b):
        @pl.loop(0, EK, step=16)
        def _(j):
            sdv[b][pl.ds(j, 16)] = dv[r][pl.ds(j, 16)]

        @plsc.parallel_loop(0, EK, 1, unroll=8)
        def _(e):
            t = hb[b][e, FG:HW] + db[b][e, 0:16]
            ex = jnp.exp(jnp.maximum(t, 0.2 * t))
            mb[b][e, 20:36] = ex
            mb[b][e, 0:16] = hb[b][e, 0:16] * _bcast(ex, i0v)
            mb[b][e, 16:32] = hb[b][e, 16:32] * _bcast(ex, i1v)

    def C0(r, b):
        g1, g2 = gath(r, b)
        g1.wait()
        g2.wait()
        compute(r, b)
        pltpu.async_copy(mb[b], spm.at[sdv[b]], ss[b], add=True)

    def C(r, b):
        g1, g2 = gath(r, b)
        g1.wait()
        g2.wait()
        pltpu.make_async_copy(mb[b], spm.at[sdv[b]], ss[b]).wait()
        compute(r, b)
        pltpu.async_copy(mb[b], spm.at[sdv[b]], ss[b], add=True)

    L(0, 0)
    L(1, 1)
    L(2, 2)
    L(3, 3)
    G(0, 0, 0)
    G(1, 1, 1)
    C0(0, 0)
    L(4, 0)
    G(2, 2, 0)
    C0(1, 1)
    L(5, 1)
    G(3, 3, 1)

    @pl.loop(0, 154)
    def _(k):
        c = 4 * k + 2
        C(2, 0)
        L(c + 4, 2)
        G(c + 2, 0, 0)
        C(3, 1)
        L(c + 5, 3)
        G(c + 3, 1, 1)
        C(0, 0)
        L(c + 6, 0)
        G(c + 4, 2, 0)
        C(1, 1)
        L(c + 7, 1)
        G(c + 5, 3, 1)

    C(2, 0)
    L(622, 2)
    G(620, 0, 0)
    C(3, 1)
    L(623, 3)
    G(621, 1, 1)
    C(0, 0)
    L(624, 0)
    G(622, 2, 0)
    C(1, 1)
    G(623, 3, 1)
    C(2, 0)
    G(624, 0, 0)
    C(3, 1)
    C(0, 0)
    pltpu.make_async_copy(mb[0], spm.at[sdv[0]], ss[0]).wait()
    pltpu.make_async_copy(mb[1], spm.at[sdv[1]], ss[1]).wait()

    plsc.subcore_barrier()

    @pl.when(s < 15)
    def _():
        pltpu.sync_copy(spm.at[pl.ds(s * NPT_A, NPT_A)],
                        out_hbm.at[pl.ds(gN + s * NPT_A, NPT_A)])

    @pl.when(s == 15)
    def _():
        pltpu.sync_copy(spm.at[pl.ds(15 * NPT_A, NPT_B)],
                        out_hbm.at[pl.ds(gN + 15 * NPT_A, NPT_B)])


@functools.partial(jax.jit, static_argnums=())
def _sc_edge_pass(ht2, ad_t, src, dst, z36):
    mesh = plsc.VectorSubcoreMesh(core_axis_name="c", subcore_axis_name="s")
    cp = pltpu.CompilerParams(needs_layout_passes=False,
                              use_tc_tiling_on_sc=False)
    k = pl.kernel(
        _sc_body,
        out_type=jax.ShapeDtypeStruct((G * N, AW), _f32),
        mesh=mesh,
        compiler_params=cp,
        scratch_types=(
            [pltpu.VMEM_SHARED((N, AW), _f32)]
            + [pltpu.VMEM((EK,), jnp.int32) for _ in range(8)]
            + [pltpu.VMEM((EK,), jnp.int32) for _ in range(4)]
            + [pltpu.VMEM((EK, 16), _f32),
               pltpu.VMEM((EK, HW), _f32), pltpu.VMEM((EK, AW), _f32)] * 2
            + [pltpu.SemaphoreType.DMA for _ in range(8)]
        ),
    )
    return k(ht2, ad_t, src, dst, z36)


def _att_mat(a_s, a_d):
    eye = jnp.eye(H, dtype=_f32)
    mk = lambda a: (eye[:, None, :] * a[:, :, None]).reshape(F, H)
    return jnp.concatenate([mk(a_s), mk(a_d)], axis=1)


def kernel(x_node_h, x_global_features, edge_index, batch_idx,
           W1, as1, ad1, b1, W2, as2, ad2, b2, W3, as3, ad3, b3,
           Wm1, bm1, Wm2, bm2, Wm3, bm3, Wm4, bm4):
    src = edge_index[0]
    dst = edge_index[1]
    bidx2 = batch_idx.reshape(N, 1)
    z36 = jnp.zeros((NPT_A, AW), _f32)
    r = jnp.repeat(jnp.eye(4, dtype=_f32), C, axis=1)

    a1 = _att_mat(as1, ad1)
    a2 = _att_mat(as2, ad2)
    a3 = _att_mat(as3, ad3)

    ht, ad_t = _stage0(x_node_h, bidx2, x_global_features, W1, a1)
    acc = _sc_edge_pass(ht.reshape(G * N, HW), ad_t, src, dst, z36)

    ht, ad_t = _mid(acc.reshape(G, N, AW), r, b1.reshape(1, F), W2, a2)
    acc = _sc_edge_pass(ht.reshape(G * N, HW), ad_t, src, dst, z36)

    ht, ad_t = _mid(acc.reshape(G, N, AW), r, b2.reshape(1, F), W3, a3)
    acc = _sc_edge_pass(ht.reshape(G * N, HW), ad_t, src, dst, z36)

    out = _stage3(acc.reshape(G, N, AW), r, b3.reshape(1, F), bidx2,
                  Wm1, bm1.reshape(1, F), Wm2, bm2.reshape(1, F),
                  Wm3, bm3.reshape(1, F), Wm4, bm4.reshape(1, 1))
    return out[:B, 0]
